# Initial kernel scaffold; baseline (speedup 1.0000x reference)
import jax
import jax.numpy as jnp
from jax.experimental import pallas as pl


def kernel(inputs, pos_table):
    raise NotImplementedError("write your pallas kernel here")


# speedup vs baseline: 1.7167x; 1.7167x over previous
import jax
import jax.numpy as jnp
from jax.experimental import pallas as pl

_BLK = 256


def _add_kernel(in_ref, tab_ref, out_ref):
    out_ref[...] = in_ref[...] + tab_ref[...][None, :, :]


def kernel(inputs, pos_table):
    batch, seq_len, embed = inputs.shape
    grid = (seq_len // _BLK,)
    return pl.pallas_call(
        _add_kernel,
        grid=grid,
        in_specs=[
            pl.BlockSpec((batch, _BLK, embed), lambda i: (0, i, 0)),
            pl.BlockSpec((_BLK, embed), lambda i: (i, 0)),
        ],
        out_specs=pl.BlockSpec((batch, _BLK, embed), lambda i: (0, i, 0)),
        out_shape=jax.ShapeDtypeStruct((batch, seq_len, embed), inputs.dtype),
    )(inputs, pos_table)


# speedup vs baseline: 1.7277x; 1.0064x over previous
import jax
import jax.numpy as jnp
from jax.experimental import pallas as pl

_BLK = 512


def _add_kernel(in_ref, tab_ref, out_ref):
    out_ref[...] = in_ref[...] + tab_ref[...][None, :, :]


def kernel(inputs, pos_table):
    batch, seq_len, embed = inputs.shape
    grid = (seq_len // _BLK,)
    return pl.pallas_call(
        _add_kernel,
        grid=grid,
        in_specs=[
            pl.BlockSpec((batch, _BLK, embed), lambda i: (0, i, 0)),
            pl.BlockSpec((_BLK, embed), lambda i: (i, 0)),
        ],
        out_specs=pl.BlockSpec((batch, _BLK, embed), lambda i: (0, i, 0)),
        out_shape=jax.ShapeDtypeStruct((batch, seq_len, embed), inputs.dtype),
    )(inputs, pos_table)
